# baseline (speedup 1.0000x reference)
import functools

import jax
import jax.numpy as jnp
from jax import lax
from jax.experimental import pallas as pl
from jax.experimental.pallas import tpu as pltpu
from jax.experimental.pallas import tpu_sc as plsc

N_BLK = 256


K_CHUNK = 2048


def _argmin_body(z_ref, cb2_ref, zn_ref, cbn_ref, idx_ref):
    zt = z_ref[0]
    zf = zt.reshape(zt.shape[0], -1).T
    zb = zf.astype(jnp.bfloat16)
    cbb = cb2_ref[...].astype(jnp.bfloat16)
    s2 = lax.dot_general(zb, cbb, (((1,), (1,)), ((), ())),
                         preferred_element_type=jnp.float32)
    d = (zn_ref[...] - s2) + cbn_ref[...]
    n, k = d.shape
    m = jnp.full((n,), jnp.inf, jnp.float32)
    mi = jnp.zeros((n,), jnp.int32)
    for c in range(0, k, K_CHUNK):
        dc = d[:, c:c + K_CHUNK]
        cmin = jnp.min(dc, axis=1)
        carg = (jnp.argmin(dc, axis=1) + c).astype(jnp.int32)
        upd = cmin < m
        mi = jnp.where(upd, carg, mi)
        m = jnp.where(upd, cmin, m).astype(jnp.bfloat16).astype(jnp.float32)
    idx_ref[0, 0, :] = mi


def _argmin_tc(z, cb2, zn, cbn):
    b, c, h, w = z.shape
    k = cb2.shape[0]
    n = b * h * w
    h_blk = N_BLK // w
    hq = h // h_blk
    num_blocks = n // N_BLK
    idx3 = pl.pallas_call(
        _argmin_body,
        grid=(b, hq),
        in_specs=[
            pl.BlockSpec((1, c, h_blk, w), lambda i, j: (i, 0, j, 0)),
            pl.BlockSpec((k, c), lambda i, j: (0, 0)),
            pl.BlockSpec((N_BLK, 1), lambda i, j: (i * hq + j, 0)),
            pl.BlockSpec((1, k), lambda i, j: (0, 0)),
        ],
        out_specs=pl.BlockSpec((1, 1, N_BLK), lambda i, j: (i * hq + j, 0, 0)),
        out_shape=jax.ShapeDtypeStruct((num_blocks, 1, N_BLK), jnp.int32),
        compiler_params=pltpu.CompilerParams(
            dimension_semantics=("parallel", "parallel")),
    )(z, cb2, zn, cbn)
    return idx3.reshape(n)


def _gather_sc(codebook_pad, idx, c_out):
    k, c_pad = codebook_pad.shape
    n = idx.shape[0]
    info = plsc.get_sparse_core_info()
    nw = info.num_cores * info.num_subcores
    b_per_w = n // nw
    mesh = plsc.VectorSubcoreMesh(core_axis_name="c", subcore_axis_name="s")

    @functools.partial(
        pl.kernel, mesh=mesh,
        out_type=jax.ShapeDtypeStruct((n, c_pad), jnp.float32),
        scratch_types=[
            pltpu.VMEM((b_per_w,), jnp.int32),
            pltpu.VMEM((b_per_w, c_pad), jnp.float32),
            pltpu.SemaphoreType.DMA,
        ],
    )
    def gather_kernel(table_hbm, idx_hbm, out_hbm, idx_v, rows_v, sem):
        wid = lax.axis_index("s") * info.num_cores + lax.axis_index("c")
        base = wid * b_per_w
        pltpu.sync_copy(idx_hbm.at[pl.ds(base, b_per_w)], idx_v)
        pltpu.async_copy(table_hbm.at[idx_v], rows_v, sem).wait()
        pltpu.sync_copy(rows_v, out_hbm.at[pl.ds(base, b_per_w)])

    return gather_kernel(codebook_pad, idx)[:, :c_out]


def kernel(z, codebook):
    b, c, h, w = z.shape
    zf = jnp.transpose(z, (0, 2, 3, 1)).reshape(-1, c)
    cb2 = 2.0 * codebook
    zn = jnp.sum(zf * zf, axis=1, keepdims=True)
    cbn = jnp.sum(codebook * codebook, axis=1)[None, :]
    idx = _argmin_tc(z, cb2, zn, cbn)
    cb_pad = jnp.pad(codebook, ((0, 0), (0, 128 - c)))
    codes_flat = _gather_sc(cb_pad, idx, c)
    codes = jnp.transpose(codes_flat.reshape(b, h, w, c), (0, 3, 1, 2))
    return (codes, idx.reshape(b, h, w))


# speedup vs baseline: 1.0512x; 1.0512x over previous
import functools

import jax
import jax.numpy as jnp
from jax import lax
from jax.experimental import pallas as pl
from jax.experimental.pallas import tpu as pltpu
from jax.experimental.pallas import tpu_sc as plsc

N_BLK = 256


K_CHUNK = 2048


def _argmin_body(zf_ref, cbt2_ref, zn_ref, cbn_ref, idx_ref):
    zb = zf_ref[...].astype(jnp.bfloat16)
    cbb = cbt2_ref[...].astype(jnp.bfloat16)
    s2 = lax.dot_general(zb, cbb, (((1,), (0,)), ((), ())),
                         preferred_element_type=jnp.float32)
    d = (zn_ref[...] - s2) + cbn_ref[...]
    n, k = d.shape
    dc = d[:, 0:K_CHUNK]
    mi = jnp.argmin(dc, axis=1).astype(jnp.int32)
    m = jnp.min(dc, axis=1).astype(jnp.bfloat16).astype(jnp.float32)
    for c in range(K_CHUNK, k, K_CHUNK):
        dc = d[:, c:c + K_CHUNK]
        cmin = jnp.min(dc, axis=1)
        carg = (jnp.argmin(dc, axis=1) + c).astype(jnp.int32)
        upd = cmin < m
        mi = jnp.where(upd, carg, mi)
        m = jnp.where(upd, cmin, m).astype(jnp.bfloat16).astype(jnp.float32)
    idx_ref[0, 0, :] = mi


def _argmin_tc(zf, cbt2, zn, cbn):
    n, c = zf.shape
    k = cbt2.shape[1]
    num_blocks = n // N_BLK
    idx3 = pl.pallas_call(
        _argmin_body,
        grid=(num_blocks,),
        in_specs=[
            pl.BlockSpec((N_BLK, c), lambda i: (i, 0)),
            pl.BlockSpec((c, k), lambda i: (0, 0)),
            pl.BlockSpec((N_BLK, 1), lambda i: (i, 0)),
            pl.BlockSpec((1, k), lambda i: (0, 0)),
        ],
        out_specs=pl.BlockSpec((1, 1, N_BLK), lambda i: (i, 0, 0)),
        out_shape=jax.ShapeDtypeStruct((num_blocks, 1, N_BLK), jnp.int32),
        compiler_params=pltpu.CompilerParams(
            dimension_semantics=("parallel",)),
    )(zf, cbt2, zn, cbn)
    return idx3.reshape(n)


def _gather_sc(codebook_pad, idx, c_out):
    k, c_pad = codebook_pad.shape
    n = idx.shape[0]
    info = plsc.get_sparse_core_info()
    nw = info.num_cores * info.num_subcores
    b_per_w = n // nw
    mesh = plsc.VectorSubcoreMesh(core_axis_name="c", subcore_axis_name="s")

    @functools.partial(
        pl.kernel, mesh=mesh,
        out_type=jax.ShapeDtypeStruct((n, c_pad), jnp.float32),
        scratch_types=[
            pltpu.VMEM((b_per_w,), jnp.int32),
            pltpu.VMEM((b_per_w, c_pad), jnp.float32),
            pltpu.SemaphoreType.DMA,
        ],
    )
    def gather_kernel(table_hbm, idx_hbm, out_hbm, idx_v, rows_v, sem):
        wid = lax.axis_index("s") * info.num_cores + lax.axis_index("c")
        base = wid * b_per_w
        pltpu.sync_copy(idx_hbm.at[pl.ds(base, b_per_w)], idx_v)
        pltpu.async_copy(table_hbm.at[idx_v], rows_v, sem).wait()
        pltpu.sync_copy(rows_v, out_hbm.at[pl.ds(base, b_per_w)])

    return gather_kernel(codebook_pad, idx)[:, :c_out]


def kernel(z, codebook):
    b, c, h, w = z.shape
    zf = jnp.transpose(z, (0, 2, 3, 1)).reshape(-1, c)
    cbt2 = 2.0 * codebook.T
    zn = jnp.sum(zf * zf, axis=1, keepdims=True)
    cbn = jnp.sum(codebook * codebook, axis=1)[None, :]
    idx = _argmin_tc(zf, cbt2, zn, cbn)
    cb_pad = jnp.pad(codebook, ((0, 0), (0, 128 - c)))
    codes_flat = _gather_sc(cb_pad, idx, c)
    codes = jnp.transpose(codes_flat.reshape(b, h, w, c), (0, 3, 1, 2))
    return (codes, idx.reshape(b, h, w))


# speedup vs baseline: 1.0537x; 1.0024x over previous
import functools

import jax
import jax.numpy as jnp
from jax import lax
from jax.experimental import pallas as pl
from jax.experimental.pallas import tpu as pltpu
from jax.experimental.pallas import tpu_sc as plsc

N_BLK = 512


K_CHUNK = 2048


def _argmin_body(zf_ref, cbt2_ref, zn_ref, cbn_ref, idx_ref):
    zb = zf_ref[...].astype(jnp.bfloat16)
    cbb = cbt2_ref[...].astype(jnp.bfloat16)
    s2 = lax.dot_general(zb, cbb, (((1,), (0,)), ((), ())),
                         preferred_element_type=jnp.float32)
    d = (zn_ref[...] - s2) + cbn_ref[...]
    n, k = d.shape
    dc = d[:, 0:K_CHUNK]
    mi = jnp.argmin(dc, axis=1).astype(jnp.int32)
    m = jnp.min(dc, axis=1).astype(jnp.bfloat16).astype(jnp.float32)
    for c in range(K_CHUNK, k, K_CHUNK):
        dc = d[:, c:c + K_CHUNK]
        cmin = jnp.min(dc, axis=1)
        carg = (jnp.argmin(dc, axis=1) + c).astype(jnp.int32)
        upd = cmin < m
        mi = jnp.where(upd, carg, mi)
        m = jnp.where(upd, cmin, m).astype(jnp.bfloat16).astype(jnp.float32)
    idx_ref[0, 0, :] = mi


def _argmin_tc(zf, cbt2, zn, cbn):
    n, c = zf.shape
    k = cbt2.shape[1]
    num_blocks = n // N_BLK
    idx3 = pl.pallas_call(
        _argmin_body,
        grid=(num_blocks,),
        in_specs=[
            pl.BlockSpec((N_BLK, c), lambda i: (i, 0)),
            pl.BlockSpec((c, k), lambda i: (0, 0)),
            pl.BlockSpec((N_BLK, 1), lambda i: (i, 0)),
            pl.BlockSpec((1, k), lambda i: (0, 0)),
        ],
        out_specs=pl.BlockSpec((1, 1, N_BLK), lambda i: (i, 0, 0)),
        out_shape=jax.ShapeDtypeStruct((num_blocks, 1, N_BLK), jnp.int32),
        compiler_params=pltpu.CompilerParams(
            dimension_semantics=("parallel",)),
    )(zf, cbt2, zn, cbn)
    return idx3.reshape(n)


def _gather_sc(codebook_pad, idx, c_out):
    k, c_pad = codebook_pad.shape
    n = idx.shape[0]
    info = plsc.get_sparse_core_info()
    nw = info.num_cores * info.num_subcores
    b_per_w = n // nw
    mesh = plsc.VectorSubcoreMesh(core_axis_name="c", subcore_axis_name="s")

    @functools.partial(
        pl.kernel, mesh=mesh,
        out_type=jax.ShapeDtypeStruct((n, c_pad), jnp.float32),
        scratch_types=[
            pltpu.VMEM((b_per_w,), jnp.int32),
            pltpu.VMEM((b_per_w, c_pad), jnp.float32),
            pltpu.SemaphoreType.DMA,
        ],
    )
    def gather_kernel(table_hbm, idx_hbm, out_hbm, idx_v, rows_v, sem):
        wid = lax.axis_index("s") * info.num_cores + lax.axis_index("c")
        base = wid * b_per_w
        pltpu.sync_copy(idx_hbm.at[pl.ds(base, b_per_w)], idx_v)
        pltpu.async_copy(table_hbm.at[idx_v], rows_v, sem).wait()
        pltpu.sync_copy(rows_v, out_hbm.at[pl.ds(base, b_per_w)])

    return gather_kernel(codebook_pad, idx)[:, :c_out]


def kernel(z, codebook):
    b, c, h, w = z.shape
    zf = jnp.transpose(z, (0, 2, 3, 1)).reshape(-1, c)
    cbt2 = 2.0 * codebook.T
    zn = jnp.sum(zf * zf, axis=1, keepdims=True)
    cbn = jnp.sum(codebook * codebook, axis=1)[None, :]
    idx = _argmin_tc(zf, cbt2, zn, cbn)
    cb_pad = jnp.pad(codebook, ((0, 0), (0, 128 - c)))
    codes_flat = _gather_sc(cb_pad, idx, c)
    codes = jnp.transpose(codes_flat.reshape(b, h, w, c), (0, 3, 1, 2))
    return (codes, idx.reshape(b, h, w))


# speedup vs baseline: 1.1149x; 1.0581x over previous
import functools

import jax
import jax.numpy as jnp
from jax import lax
from jax.experimental import pallas as pl
from jax.experimental.pallas import tpu as pltpu
from jax.experimental.pallas import tpu_sc as plsc

N_BLK = 512


K_CHUNK = 2048


def _argmin_body(zf_ref, cbt2_ref, cbn_ref, idx_ref):
    zf = zf_ref[...]
    zb = zf.astype(jnp.bfloat16)
    cbb = cbt2_ref[...].astype(jnp.bfloat16)
    s2 = lax.dot_general(zb, cbb, (((1,), (0,)), ((), ())),
                         preferred_element_type=jnp.float32)
    zn = jnp.sum(zf * zf, axis=1, keepdims=True)
    d = (zn - s2) + cbn_ref[...]
    n, k = d.shape
    dc = d[:, 0:K_CHUNK]
    mi = jnp.argmin(dc, axis=1).astype(jnp.int32)
    m = jnp.min(dc, axis=1).astype(jnp.bfloat16).astype(jnp.float32)
    for c in range(K_CHUNK, k, K_CHUNK):
        dc = d[:, c:c + K_CHUNK]
        cmin = jnp.min(dc, axis=1)
        carg = (jnp.argmin(dc, axis=1) + c).astype(jnp.int32)
        upd = cmin < m
        mi = jnp.where(upd, carg, mi)
        m = jnp.where(upd, cmin, m).astype(jnp.bfloat16).astype(jnp.float32)
    idx_ref[0, 0, :] = mi


def _argmin_tc(zf, cbt2, cbn):
    n, c = zf.shape
    k = cbt2.shape[1]
    num_blocks = n // N_BLK
    idx3 = pl.pallas_call(
        _argmin_body,
        grid=(num_blocks,),
        in_specs=[
            pl.BlockSpec((N_BLK, c), lambda i: (i, 0)),
            pl.BlockSpec((c, k), lambda i: (0, 0)),
            pl.BlockSpec((1, k), lambda i: (0, 0)),
        ],
        out_specs=pl.BlockSpec((1, 1, N_BLK), lambda i: (i, 0, 0)),
        out_shape=jax.ShapeDtypeStruct((num_blocks, 1, N_BLK), jnp.int32),
        compiler_params=pltpu.CompilerParams(
            dimension_semantics=("parallel",)),
    )(zf, cbt2, cbn)
    return idx3.reshape(n)


def _gather_sc(codebook_pad, idx, c_out):
    k, c_pad = codebook_pad.shape
    n = idx.shape[0]
    info = plsc.get_sparse_core_info()
    nw = info.num_cores * info.num_subcores
    b_per_w = n // nw
    mesh = plsc.VectorSubcoreMesh(core_axis_name="c", subcore_axis_name="s")

    @functools.partial(
        pl.kernel, mesh=mesh,
        out_type=jax.ShapeDtypeStruct((n, c_pad), jnp.float32),
        scratch_types=[
            pltpu.VMEM((b_per_w,), jnp.int32),
            pltpu.VMEM((b_per_w, c_pad), jnp.float32),
            pltpu.SemaphoreType.DMA,
        ],
    )
    def gather_kernel(table_hbm, idx_hbm, out_hbm, idx_v, rows_v, sem):
        wid = lax.axis_index("s") * info.num_cores + lax.axis_index("c")
        base = wid * b_per_w
        pltpu.sync_copy(idx_hbm.at[pl.ds(base, b_per_w)], idx_v)
        pltpu.async_copy(table_hbm.at[idx_v], rows_v, sem).wait()
        pltpu.sync_copy(rows_v, out_hbm.at[pl.ds(base, b_per_w)])

    return gather_kernel(codebook_pad, idx)[:, :c_out]


def kernel(z, codebook):
    b, c, h, w = z.shape
    zf = jnp.transpose(z, (0, 2, 3, 1)).reshape(-1, c)
    cbt2 = 2.0 * codebook.T
    cbn = jnp.sum(codebook * codebook, axis=1)[None, :]
    idx = _argmin_tc(zf, cbt2, cbn)
    cb_pad = jnp.pad(codebook, ((0, 0), (0, 128 - c)))
    codes_flat = _gather_sc(cb_pad, idx, c)
    codes = jnp.transpose(codes_flat.reshape(b, h, w, c), (0, 3, 1, 2))
    return (codes, idx.reshape(b, h, w))
